# baseline (speedup 1.0000x reference)
import functools

import jax
import jax.numpy as jnp
from jax import lax
from jax.experimental import pallas as pl
from jax.experimental.pallas import tpu as pltpu
from jax.experimental.pallas import tpu_sc as plsc

T = 4096
D = 1024
F = 4096
E = 8
CAP = 640
ZROW = E * CAP
TRASH = ZROW + 1
NROW = TRASH + 1
GW = 128
CHUNK = 512
NCHUNK = T // CHUNK
FFT = 2048
KT = F // FFT
NW = 32
TPW = T // NW
SUB = 64
NSUB = TPW // SUB


def _route(x_flat, Wg, bg2):
    def body(x_ref, wg_ref, bg_ref, dst_ref, src_ref, g_ref, aux_ref,
             cnt_ref, carry_ref, imp_ref):
        c = pl.program_id(0)

        @pl.when(c == 0)
        def _():
            carry_ref[...] = jnp.zeros_like(carry_ref)
            imp_ref[...] = jnp.zeros_like(imp_ref)

        x = x_ref[...].astype(jnp.bfloat16)
        logits = lax.dot_general(
            x, wg_ref[...].astype(jnp.bfloat16), (((1,), (1,)), ((), ())),
            preferred_element_type=jnp.float32) + bg_ref[...]
        m = jnp.max(logits, axis=1, keepdims=True)
        ex = jnp.exp(logits - m)
        p = ex / jnp.sum(ex, axis=1, keepdims=True)

        raw = (logits == m).astype(jnp.float32)
        m8 = (lax.broadcasted_iota(jnp.int32, (E, E), 0)
              <= lax.broadcasted_iota(jnp.int32, (E, E), 1)).astype(jnp.float32)
        cum8 = lax.dot_general(raw.astype(jnp.bfloat16),
                               m8.astype(jnp.bfloat16),
                               (((1,), (0,)), ((), ())),
                               preferred_element_type=jnp.float32)
        onehot = jnp.where((cum8 == 1.0) & (raw == 1.0), 1.0, 0.0)

        tril = (lax.broadcasted_iota(jnp.int32, (CHUNK, CHUNK), 1)
                <= lax.broadcasted_iota(jnp.int32, (CHUNK, CHUNK), 0)
                ).astype(jnp.float32)
        within = lax.dot_general(tril.astype(jnp.bfloat16),
                                 onehot.astype(jnp.bfloat16),
                                 (((1,), (0,)), ((), ())),
                                 preferred_element_type=jnp.float32)
        tot = within + carry_ref[...]

        iota8 = lax.broadcasted_iota(jnp.int32, (1, E), 1).astype(jnp.float32)
        top_idx = jnp.sum(onehot * iota8, axis=1, keepdims=True)
        top_gate = jnp.sum(p * onehot, axis=1, keepdims=True)
        pos = jnp.sum(tot * onehot, axis=1, keepdims=True) - 1.0
        kept = pos < float(CAP)

        dst = jnp.where(kept, top_idx * CAP + pos, float(TRASH))
        src = jnp.where(kept, top_idx * CAP + pos, float(ZROW))
        dst_ref[...] = dst.astype(jnp.int32).reshape(CHUNK // SUB, SUB)
        src_ref[...] = src.astype(jnp.int32).reshape(CHUNK // SUB, SUB)
        g_ref[...] = jnp.broadcast_to(jnp.where(kept, top_gate, 0.0),
                                      (CHUNK, GW))

        carry_ref[...] += jnp.sum(onehot, axis=0, keepdims=True)
        imp_ref[...] += jnp.sum(p, axis=0, keepdims=True)

        @pl.when(c == NCHUNK - 1)
        def _():
            aux_ref[...] = (jnp.sum(imp_ref[...] * carry_ref[...],
                                    axis=1, keepdims=True)
                            * (float(E) / (float(T) * float(T))))
            cnt_ref[...] = jnp.minimum(carry_ref[...],
                                       float(CAP)).astype(jnp.int32)

    return pl.pallas_call(
        body,
        grid=(NCHUNK,),
        in_specs=[
            pl.BlockSpec((CHUNK, D), lambda c: (c, 0)),
            pl.BlockSpec((E, D), lambda c: (0, 0)),
            pl.BlockSpec((1, E), lambda c: (0, 0)),
        ],
        out_specs=[
            pl.BlockSpec((CHUNK // SUB, SUB), lambda c: (c, 0)),
            pl.BlockSpec((CHUNK // SUB, SUB), lambda c: (c, 0)),
            pl.BlockSpec((CHUNK, GW), lambda c: (c, 0)),
            pl.BlockSpec((1, 1), lambda c: (0, 0)),
            pl.BlockSpec((1, E), lambda c: (0, 0)),
        ],
        out_shape=[
            jax.ShapeDtypeStruct((T // SUB, SUB), jnp.int32),
            jax.ShapeDtypeStruct((T // SUB, SUB), jnp.int32),
            jax.ShapeDtypeStruct((T, GW), jnp.float32),
            jax.ShapeDtypeStruct((1, 1), jnp.float32),
            jax.ShapeDtypeStruct((1, E), jnp.int32),
        ],
        scratch_shapes=[
            pltpu.VMEM((1, E), jnp.float32),
            pltpu.VMEM((1, E), jnp.float32),
        ],
        compiler_params=pltpu.CompilerParams(
            dimension_semantics=("arbitrary",)),
    )(x_flat, Wg, bg2)


def _dispatch_sc(x_flat, g16, dst2d):
    mesh = plsc.VectorSubcoreMesh(core_axis_name="c", subcore_axis_name="s")

    @functools.partial(
        pl.kernel,
        out_type=[
            jax.ShapeDtypeStruct((NROW, D), jnp.float32),
            jax.ShapeDtypeStruct((NROW, GW), jnp.float32),
        ],
        mesh=mesh,
        scratch_types=[
            pltpu.VMEM((NSUB, SUB), jnp.int32),
            pltpu.VMEM((SUB, D), jnp.float32),
            pltpu.VMEM((SUB, GW), jnp.float32),
            pltpu.SemaphoreType.DMA,
        ],
    )
    def disp(x_hbm, g_hbm, d_hbm, buf_hbm, gbuf_hbm, idx_v, rows_v, grow_v,
             sem):
        wid = lax.axis_index("s") * 2 + lax.axis_index("c")
        pltpu.async_copy(d_hbm.at[pl.ds(wid * NSUB, NSUB)], idx_v, sem).wait()

        @pl.when(wid == 0)
        def _():
            @pl.loop(0, D, step=16)
            def _(i):
                rows_v.at[0, pl.ds(i, 16)][...] = jnp.zeros((16,), jnp.float32)
            pltpu.async_copy(rows_v.at[pl.ds(0, 1)],
                             buf_hbm.at[pl.ds(ZROW, 1)], sem).wait()

        for j in range(NSUB):
            base = wid * TPW + j * SUB
            pltpu.async_copy(x_hbm.at[pl.ds(base, SUB)], rows_v, sem).wait()
            pltpu.async_copy(rows_v, buf_hbm.at[idx_v.at[j]], sem).wait()
            pltpu.async_copy(g_hbm.at[pl.ds(base, SUB)], grow_v, sem).wait()
            pltpu.async_copy(grow_v, gbuf_hbm.at[idx_v.at[j]], sem).wait()

    return disp(x_flat, g16, dst2d)


def _combine_sc(eo, src2d):
    mesh = plsc.VectorSubcoreMesh(core_axis_name="c", subcore_axis_name="s")

    @functools.partial(
        pl.kernel,
        out_type=jax.ShapeDtypeStruct((T, D), jnp.float32),
        mesh=mesh,
        scratch_types=[
            pltpu.VMEM((NSUB, SUB), jnp.int32),
            pltpu.VMEM((SUB, D), jnp.float32),
            pltpu.SemaphoreType.DMA,
        ],
    )
    def comb(eo_hbm, s_hbm, out_hbm, idx_v, rows_v, sem):
        wid = lax.axis_index("s") * 2 + lax.axis_index("c")
        pltpu.async_copy(s_hbm.at[pl.ds(wid * NSUB, NSUB)], idx_v, sem).wait()
        for j in range(NSUB):
            base = wid * TPW + j * SUB
            pltpu.async_copy(eo_hbm.at[idx_v.at[j]], rows_v, sem).wait()
            pltpu.async_copy(rows_v, out_hbm.at[pl.ds(base, SUB)], sem).wait()

    return comb(eo, src2d)


RT = 128
NR = CAP // RT


def _ffn(buf, gbuf, cnt, W1, b1_3d, W2, b2_3d):
    def body(cnt_ref, x_ref, g_ref, w1_ref, b1_ref, w2_ref, b2_ref, o_ref):
        e = pl.program_id(0)
        k = pl.program_id(1)
        r = pl.program_id(2)
        rows = pl.ds(r * RT, RT)

        @pl.when(r * RT < cnt_ref[0, e])
        def _():
            x = x_ref[rows, :].astype(jnp.bfloat16)
            w1 = w1_ref[0].astype(jnp.bfloat16)
            h = lax.dot_general(x, w1, (((1,), (1,)), ((), ())),
                                preferred_element_type=jnp.float32)
            h = jnp.maximum(h + b1_ref[0], 0.0).astype(jnp.bfloat16)
            w2 = w2_ref[0].astype(jnp.bfloat16)
            contrib = lax.dot_general(h, w2, (((1,), (1,)), ((), ())),
                                      preferred_element_type=jnp.float32)

            @pl.when(k == 0)
            def _():
                o_ref[rows, :] = contrib

            @pl.when((k > 0) & (k < KT - 1))
            def _():
                o_ref[rows, :] += contrib

            @pl.when(k == KT - 1)
            def _():
                gcol = g_ref[rows, :][:, :1]
                o_ref[rows, :] = (o_ref[rows, :] + contrib
                                  + b2_ref[0]) * gcol

    return pl.pallas_call(
        body,
        grid=(E, KT, NR),
        in_specs=[
            pl.BlockSpec(memory_space=pltpu.SMEM),
            pl.BlockSpec((CAP, D), lambda e, k, r: (e, 0)),
            pl.BlockSpec((CAP, GW), lambda e, k, r: (e, 0)),
            pl.BlockSpec((1, FFT, D), lambda e, k, r: (e, k, 0)),
            pl.BlockSpec((1, 1, FFT), lambda e, k, r: (e, 0, k)),
            pl.BlockSpec((1, D, FFT), lambda e, k, r: (e, 0, k)),
            pl.BlockSpec((1, 1, D), lambda e, k, r: (e, 0, 0)),
        ],
        out_specs=pl.BlockSpec((CAP, D), lambda e, k, r: (e, 0)),
        out_shape=jax.ShapeDtypeStruct((NROW, D), jnp.float32),
        input_output_aliases={1: 0},
        compiler_params=pltpu.CompilerParams(
            dimension_semantics=("arbitrary", "arbitrary", "arbitrary")),
    )(cnt, buf, gbuf, W1, b1_3d, W2, b2_3d)


def kernel(x, Wg, bg, W1, b1, W2, b2):
    Bb, Ss, Dd = x.shape
    x_flat = x.reshape(T, D)
    bg2 = bg.reshape(1, E)
    dst2d, src2d, g16, aux, cnt = _route(x_flat, Wg, bg2)
    buf, gbuf = _dispatch_sc(x_flat, g16, dst2d)
    eo = _ffn(buf, gbuf, cnt, W1, b1.reshape(E, 1, F), W2,
              b2.reshape(E, 1, D))
    out_flat = _combine_sc(eo, src2d)
    return out_flat.reshape(Bb, Ss, Dd), aux[0, 0]


# speedup vs baseline: 2.2112x; 2.2112x over previous
import functools

import jax
import jax.numpy as jnp
from jax import lax
from jax.experimental import pallas as pl
from jax.experimental.pallas import tpu as pltpu
from jax.experimental.pallas import tpu_sc as plsc

T = 4096
D = 1024
F = 4096
E = 8
CAP = 640
ZROW = E * CAP
TRASH = ZROW + 1
NROW = TRASH + 1
GW = 128
CHUNK = 512
NCHUNK = T // CHUNK
FFT = 2048
KT = F // FFT
NW = 32
TPW = T // NW
SUB = 64
NSUB = TPW // SUB


def _route(x_flat, Wg, bg2):
    def body(x_ref, wg_ref, bg_ref, dst_ref, src_ref, g_ref, aux_ref,
             cnt_ref, carry_ref, imp_ref):
        c = pl.program_id(0)

        @pl.when(c == 0)
        def _():
            carry_ref[...] = jnp.zeros_like(carry_ref)
            imp_ref[...] = jnp.zeros_like(imp_ref)

        x = x_ref[...].astype(jnp.bfloat16)
        logits = lax.dot_general(
            x, wg_ref[...].astype(jnp.bfloat16), (((1,), (1,)), ((), ())),
            preferred_element_type=jnp.float32) + bg_ref[...]
        m = jnp.max(logits, axis=1, keepdims=True)
        ex = jnp.exp(logits - m)
        p = ex / jnp.sum(ex, axis=1, keepdims=True)

        raw = (logits == m).astype(jnp.float32)
        m8 = (lax.broadcasted_iota(jnp.int32, (E, E), 0)
              <= lax.broadcasted_iota(jnp.int32, (E, E), 1)).astype(jnp.float32)
        cum8 = lax.dot_general(raw.astype(jnp.bfloat16),
                               m8.astype(jnp.bfloat16),
                               (((1,), (0,)), ((), ())),
                               preferred_element_type=jnp.float32)
        onehot = jnp.where((cum8 == 1.0) & (raw == 1.0), 1.0, 0.0)

        tril = (lax.broadcasted_iota(jnp.int32, (CHUNK, CHUNK), 1)
                <= lax.broadcasted_iota(jnp.int32, (CHUNK, CHUNK), 0)
                ).astype(jnp.float32)
        within = lax.dot_general(tril.astype(jnp.bfloat16),
                                 onehot.astype(jnp.bfloat16),
                                 (((1,), (0,)), ((), ())),
                                 preferred_element_type=jnp.float32)
        tot = within + carry_ref[...]

        iota8 = lax.broadcasted_iota(jnp.int32, (1, E), 1).astype(jnp.float32)
        top_idx = jnp.sum(onehot * iota8, axis=1, keepdims=True)
        top_gate = jnp.sum(p * onehot, axis=1, keepdims=True)
        pos = jnp.sum(tot * onehot, axis=1, keepdims=True) - 1.0
        kept = pos < float(CAP)

        dst = jnp.where(kept, top_idx * CAP + pos, float(TRASH))
        src = jnp.where(kept, top_idx * CAP + pos, float(ZROW))
        dst_ref[...] = dst.astype(jnp.int32).reshape(CHUNK // SUB, SUB)
        src_ref[...] = src.astype(jnp.int32).reshape(CHUNK // SUB, SUB)
        g_ref[...] = jnp.broadcast_to(jnp.where(kept, top_gate, 0.0),
                                      (CHUNK, GW))

        carry_ref[...] += jnp.sum(onehot, axis=0, keepdims=True)
        imp_ref[...] += jnp.sum(p, axis=0, keepdims=True)

        @pl.when(c == NCHUNK - 1)
        def _():
            aux_ref[...] = (jnp.sum(imp_ref[...] * carry_ref[...],
                                    axis=1, keepdims=True)
                            * (float(E) / (float(T) * float(T))))
            cnt_ref[...] = jnp.minimum(carry_ref[...],
                                       float(CAP)).astype(jnp.int32)

    return pl.pallas_call(
        body,
        grid=(NCHUNK,),
        in_specs=[
            pl.BlockSpec((CHUNK, D), lambda c: (c, 0)),
            pl.BlockSpec((E, D), lambda c: (0, 0)),
            pl.BlockSpec((1, E), lambda c: (0, 0)),
        ],
        out_specs=[
            pl.BlockSpec((CHUNK // SUB, SUB), lambda c: (c, 0)),
            pl.BlockSpec((CHUNK // SUB, SUB), lambda c: (c, 0)),
            pl.BlockSpec((CHUNK, GW), lambda c: (c, 0)),
            pl.BlockSpec((1, 1), lambda c: (0, 0)),
            pl.BlockSpec((1, E), lambda c: (0, 0)),
        ],
        out_shape=[
            jax.ShapeDtypeStruct((T // SUB, SUB), jnp.int32),
            jax.ShapeDtypeStruct((T // SUB, SUB), jnp.int32),
            jax.ShapeDtypeStruct((T, GW), jnp.float32),
            jax.ShapeDtypeStruct((1, 1), jnp.float32),
            jax.ShapeDtypeStruct((1, E), jnp.int32),
        ],
        scratch_shapes=[
            pltpu.VMEM((1, E), jnp.float32),
            pltpu.VMEM((1, E), jnp.float32),
        ],
        compiler_params=pltpu.CompilerParams(
            dimension_semantics=("arbitrary",)),
    )(x_flat, Wg, bg2)


def _dispatch_sc(x_flat, g16, dst2d):
    mesh = plsc.VectorSubcoreMesh(core_axis_name="c", subcore_axis_name="s")

    @functools.partial(
        pl.kernel,
        out_type=[
            jax.ShapeDtypeStruct((NROW, D), jnp.float32),
            jax.ShapeDtypeStruct((NROW, GW), jnp.float32),
        ],
        mesh=mesh,
        scratch_types=[
            pltpu.VMEM((NSUB, SUB), jnp.int32),
            pltpu.VMEM((SUB, D), jnp.float32),
            pltpu.VMEM((SUB, GW), jnp.float32),
            pltpu.SemaphoreType.DMA,
        ],
    )
    def disp(x_hbm, g_hbm, d_hbm, buf_hbm, gbuf_hbm, idx_v, rows_v, grow_v,
             sem):
        wid = lax.axis_index("s") * 2 + lax.axis_index("c")
        pltpu.async_copy(d_hbm.at[pl.ds(wid * NSUB, NSUB)], idx_v, sem).wait()

        @pl.when(wid == 0)
        def _():
            @pl.loop(0, D, step=16)
            def _(i):
                rows_v.at[0, pl.ds(i, 16)][...] = jnp.zeros((16,), jnp.float32)
            pltpu.async_copy(rows_v.at[pl.ds(0, 1)],
                             buf_hbm.at[pl.ds(ZROW, 1)], sem).wait()

        for j in range(NSUB):
            base = wid * TPW + j * SUB
            pltpu.async_copy(x_hbm.at[pl.ds(base, SUB)], rows_v, sem).wait()
            pltpu.async_copy(rows_v, buf_hbm.at[idx_v.at[j]], sem).wait()
            pltpu.async_copy(g_hbm.at[pl.ds(base, SUB)], grow_v, sem).wait()
            pltpu.async_copy(grow_v, gbuf_hbm.at[idx_v.at[j]], sem).wait()

    return disp(x_flat, g16, dst2d)


def _combine_sc(eo, src2d):
    mesh = plsc.VectorSubcoreMesh(core_axis_name="c", subcore_axis_name="s")

    @functools.partial(
        pl.kernel,
        out_type=jax.ShapeDtypeStruct((T, D), jnp.float32),
        mesh=mesh,
        scratch_types=[
            pltpu.VMEM((NSUB, SUB), jnp.int32),
            pltpu.VMEM((SUB, D), jnp.float32),
            pltpu.SemaphoreType.DMA,
        ],
    )
    def comb(eo_hbm, s_hbm, out_hbm, idx_v, rows_v, sem):
        wid = lax.axis_index("s") * 2 + lax.axis_index("c")
        pltpu.async_copy(s_hbm.at[pl.ds(wid * NSUB, NSUB)], idx_v, sem).wait()
        for j in range(NSUB):
            base = wid * TPW + j * SUB
            pltpu.async_copy(eo_hbm.at[idx_v.at[j]], rows_v, sem).wait()
            pltpu.async_copy(rows_v, out_hbm.at[pl.ds(base, SUB)], sem).wait()

    return comb(eo, src2d)


def _ffn(buf, gbuf, W1, b1_3d, W2, b2_3d):
    def body(x_ref, g_ref, w1_ref, b1_ref, w2_ref, b2_ref, o_ref):
        k = pl.program_id(1)
        x = x_ref[...].astype(jnp.bfloat16)
        w1 = w1_ref[0].astype(jnp.bfloat16)
        h = lax.dot_general(x, w1, (((1,), (1,)), ((), ())),
                            preferred_element_type=jnp.float32)
        h = jnp.maximum(h + b1_ref[0], 0.0).astype(jnp.bfloat16)
        w2 = w2_ref[0].astype(jnp.bfloat16)
        contrib = lax.dot_general(h, w2, (((1,), (1,)), ((), ())),
                                  preferred_element_type=jnp.float32)

        @pl.when(k == 0)
        def _():
            o_ref[...] = contrib

        @pl.when((k > 0) & (k < KT - 1))
        def _():
            o_ref[...] += contrib

        @pl.when(k == KT - 1)
        def _():
            gcol = g_ref[...][:, :1]
            o_ref[...] = (o_ref[...] + contrib + b2_ref[0]) * gcol

    return pl.pallas_call(
        body,
        grid=(E, KT),
        in_specs=[
            pl.BlockSpec((CAP, D), lambda e, k: (e, 0)),
            pl.BlockSpec((CAP, GW), lambda e, k: (e, 0)),
            pl.BlockSpec((1, FFT, D), lambda e, k: (e, k, 0)),
            pl.BlockSpec((1, 1, FFT), lambda e, k: (e, 0, k)),
            pl.BlockSpec((1, D, FFT), lambda e, k: (e, 0, k)),
            pl.BlockSpec((1, 1, D), lambda e, k: (e, 0, 0)),
        ],
        out_specs=pl.BlockSpec((CAP, D), lambda e, k: (e, 0)),
        out_shape=jax.ShapeDtypeStruct((NROW, D), jnp.float32),
        input_output_aliases={0: 0},
        compiler_params=pltpu.CompilerParams(
            dimension_semantics=("arbitrary", "arbitrary")),
    )(buf, gbuf, W1, b1_3d, W2, b2_3d)


def kernel(x, Wg, bg, W1, b1, W2, b2):
    Bb, Ss, Dd = x.shape
    x_flat = x.reshape(T, D)
    bg2 = bg.reshape(1, E)
    dst2d, src2d, g16, aux, _ = _route(x_flat, Wg, bg2)
    buf, gbuf = _dispatch_sc(x_flat, g16, dst2d)
    eo = _ffn(buf, gbuf, W1, b1.reshape(E, 1, F), W2, b2.reshape(E, 1, D))
    out_flat = _combine_sc(eo, src2d)
    return out_flat.reshape(Bb, Ss, Dd), aux[0, 0]
